# Initial kernel scaffold; baseline (speedup 1.0000x reference)
import jax
import jax.numpy as jnp
from jax.experimental import pallas as pl


def kernel(x, edge_index, edge_attr, u, batch, params):
    raise NotImplementedError("write your pallas kernel here")


# speedup vs baseline: 1.7586x; 1.7586x over previous
import functools

import jax
import jax.numpy as jnp
from jax import lax
from jax.experimental import pallas as pl
from jax.experimental.pallas import tpu as pltpu
from jax.experimental.pallas import tpu_sc as plsc

N = 10000
E = 320000
F_IN = 128
H = 64
G = 16

NC = 2
NS = 16
NW = NC * NS
EW = E // NW
CH = 80
NCHUNK = EW // CH

ZROW = 624
ZCNT = 640


def _edge_aggregate(x_tab, ea2, src, dst, zeros, F):
    mesh = plsc.VectorSubcoreMesh(core_axis_name="c", subcore_axis_name="s")

    @functools.partial(
        pl.kernel,
        mesh=mesh,
        compiler_params=pltpu.CompilerParams(use_tc_tiling_on_sc=False),
        out_type=jax.ShapeDtypeStruct((2 * N, F), jnp.float32),
        scratch_types=[
            pltpu.VMEM((CH,), jnp.int32),
            pltpu.VMEM((CH,), jnp.int32),
            pltpu.VMEM((CH, F), jnp.float32),
            pltpu.VMEM((CH, F), jnp.float32),
            pltpu.VMEM_SHARED((N, F), jnp.float32),
            pltpu.SemaphoreType.DMA,
        ],
    )
    def k(x_hbm, ea_hbm, src_hbm, dst_hbm, z_hbm, out_hbm,
          src_v, dst_v, xg_v, ea_v, aggr_sh, sem):
        cid = lax.axis_index("c")
        sid = lax.axis_index("s")
        wid = cid * NS + sid
        zbase = sid * ZROW

        pltpu.sync_copy(z_hbm.at[pl.ds(zbase, ZCNT)], aggr_sh.at[pl.ds(zbase, ZCNT)])
        plsc.subcore_barrier()

        def chunk_body(kk, carry):
            base = wid * EW + kk * CH
            pltpu.sync_copy(src_hbm.at[pl.ds(base, CH)], src_v)
            pltpu.sync_copy(dst_hbm.at[pl.ds(base, CH)], dst_v)
            pltpu.sync_copy(ea_hbm.at[pl.ds(base, CH)], ea_v)
            pltpu.async_copy(x_hbm.at[src_v], xg_v, sem).wait()

            def row_body(r, c2):
                for j in range(F // 16):
                    sl = pl.ds(j * 16, 16)
                    v = xg_v[r, sl] + ea_v[r, sl]
                    xg_v[r, sl] = jnp.maximum(v, 0.0)
                return c2

            lax.fori_loop(0, CH, row_body, 0)
            pltpu.sync_copy(xg_v, aggr_sh.at[dst_v], add=True)
            return carry

        lax.fori_loop(0, NCHUNK, chunk_body, 0)
        plsc.subcore_barrier()
        pltpu.sync_copy(aggr_sh.at[pl.ds(zbase, ZCNT)],
                        out_hbm.at[pl.ds(cid * N + zbase, ZCNT)])

    return k(x_tab, ea2, src, dst, zeros)


def _edge_transform(ea, w1a, b1a, wc1, bc1, w1b, b1b, wc2, bc2):
    BE = 3200
    grid = (E // BE,)
    const2 = lambda i: (0, 0)
    row = lambda i: (i, 0)

    def body(ea_ref, w1a_r, b1a_r, wc1_r, bc1_r, w1b_r, b1b_r, wc2_r, bc2_r,
             o1a, o1b, o1c, o2a, o2b, o2c):
        e = ea_ref[...]
        t1 = jnp.maximum(
            jnp.dot(e, w1a_r[...], preferred_element_type=jnp.float32)
            + b1a_r[0:1, :], 0.0)
        z1 = jnp.dot(t1, wc1_r[...], preferred_element_type=jnp.float32) + bc1_r[0:1, :]
        o1a[...] = z1[:, 0:128]
        o1b[...] = z1[:, 128:256]
        o1c[...] = z1[:, 256:384]
        t2 = jnp.maximum(
            jnp.dot(e, w1b_r[...], preferred_element_type=jnp.float32)
            + b1b_r[0:1, :], 0.0)
        z2 = jnp.dot(t2, wc2_r[...], preferred_element_type=jnp.float32) + bc2_r[0:1, :]
        o2a[...] = z2[:, 0:64]
        o2b[...] = z2[:, 64:128]
        o2c[...] = z2[:, 128:192]

    f32 = jnp.float32
    return pl.pallas_call(
        body,
        grid=grid,
        in_specs=[
            pl.BlockSpec((BE, 16), row),
            pl.BlockSpec((16, H), const2),
            pl.BlockSpec((8, H), const2),
            pl.BlockSpec((H, 3 * F_IN), const2),
            pl.BlockSpec((8, 3 * F_IN), const2),
            pl.BlockSpec((16, H), const2),
            pl.BlockSpec((8, H), const2),
            pl.BlockSpec((H, 3 * H), const2),
            pl.BlockSpec((8, 3 * H), const2),
        ],
        out_specs=[
            pl.BlockSpec((BE, F_IN), row),
            pl.BlockSpec((BE, F_IN), row),
            pl.BlockSpec((BE, F_IN), row),
            pl.BlockSpec((BE, H), row),
            pl.BlockSpec((BE, H), row),
            pl.BlockSpec((BE, H), row),
        ],
        out_shape=[
            jax.ShapeDtypeStruct((E, F_IN), f32),
            jax.ShapeDtypeStruct((E, F_IN), f32),
            jax.ShapeDtypeStruct((E, F_IN), f32),
            jax.ShapeDtypeStruct((E, H), f32),
            jax.ShapeDtypeStruct((E, H), f32),
            jax.ShapeDtypeStruct((E, H), f32),
        ],
    )(ea, w1a, b1a, wc1, bc1, w1b, b1b, wc2, bc2)


def _node_update(xin, pa, pb, pc, n1w, n1b, n2w, n2b, l1w, l1b, g, beta, Fin):
    BN = 1000
    grid = (N // BN,)
    row = lambda i: (i, 0)
    shift = lambda i: (i + N // BN, 0)
    const2 = lambda i: (0, 0)
    f32 = jnp.float32

    def body(x_r, pa0, pa1, pb0, pb1, pc0, pc1,
             n1wa, n1wb, n1wc, n1ba, n1bb, n1bc,
             n2wa, n2wb, n2wc, n2ba, n2bb, n2bc,
             l1w_r, l1b_r, g_r, beta_r, out_r):
        x_b = x_r[...]

        def conv(p0, p1, w1, b1, w2, b2):
            h = x_b + p0[...] + p1[...]
            t = jnp.maximum(
                jnp.dot(h, w1[...], preferred_element_type=f32) + b1[0:1, :], 0.0)
            o = jnp.dot(t, w2[...], preferred_element_type=f32) + b2[0:1, :]
            return jnp.maximum(o, 0.0)

        cat = jnp.concatenate([
            conv(pa0, pa1, n1wa, n1ba, n2wa, n2ba),
            conv(pb0, pb1, n1wb, n1bb, n2wb, n2bb),
            conv(pc0, pc1, n1wc, n1bc, n2wc, n2bc),
        ], axis=1)
        y = jnp.maximum(
            jnp.dot(cat, l1w_r[...], preferred_element_type=f32) + l1b_r[0:1, :], 0.0)
        m = jnp.mean(y, axis=-1, keepdims=True)
        v = jnp.mean((y - m) ** 2, axis=-1, keepdims=True)
        out_r[...] = (y - m) / jnp.sqrt(v + 1e-5) * g_r[0:1, :] + beta_r[0:1, :]

    in_specs = [pl.BlockSpec((BN, Fin), row)]
    for _ in range(3):
        in_specs += [pl.BlockSpec((BN, Fin), row), pl.BlockSpec((BN, Fin), shift)]
    in_specs += [pl.BlockSpec((Fin, H), const2)] * 3
    in_specs += [pl.BlockSpec((8, H), const2)] * 3
    in_specs += [pl.BlockSpec((H, H), const2)] * 3
    in_specs += [pl.BlockSpec((8, H), const2)] * 3
    in_specs += [
        pl.BlockSpec((3 * H, H), const2),
        pl.BlockSpec((8, H), const2),
        pl.BlockSpec((8, H), const2),
        pl.BlockSpec((8, H), const2),
    ]
    return pl.pallas_call(
        body,
        grid=grid,
        in_specs=in_specs,
        out_specs=pl.BlockSpec((BN, H), row),
        out_shape=jax.ShapeDtypeStruct((N, H), f32),
    )(xin, pa, pa, pb, pb, pc, pc,
      n1w[0], n1w[1], n1w[2], n1b[0], n1b[1], n1b[2],
      n2w[0], n2w[1], n2w[2], n2b[0], n2b[1], n2b[2],
      l1w, l1b, g, beta)


def _pool_head(x2, batch2d, ones_col, u, fc1w, fc1b, g3, b3, fc2w, fc2b):
    BN = 1000
    grid = (N // BN,)
    row = lambda i: (i, 0)
    const2 = lambda i: (0, 0)
    f32 = jnp.float32

    def body(x_r, b_r, one_r, u_r, w1_r, b1_r, g_r, be_r, w2_r, b2_r, out_r,
             sums, cnts):
        i = pl.program_id(0)

        @pl.when(i == 0)
        def _():
            sums[...] = jnp.zeros_like(sums)
            cnts[...] = jnp.zeros_like(cnts)

        oh = (b_r[...] == lax.broadcasted_iota(jnp.int32, (1, G), 1)).astype(f32)
        sums[...] += lax.dot_general(oh, x_r[...], (((0,), (0,)), ((), ())),
                                     preferred_element_type=f32)
        cnts[...] += lax.dot_general(oh, one_r[...], (((0,), (0,)), ((), ())),
                                     preferred_element_type=f32)

        @pl.when(i == grid[0] - 1)
        def _():
            mean = sums[...] / jnp.maximum(cnts[...], 1.0)
            xf = jnp.concatenate([mean, u_r[...]], axis=1)
            h = jnp.maximum(
                jnp.dot(xf, w1_r[...], preferred_element_type=f32) + b1_r[0:1, :],
                0.0)
            mu = jnp.mean(h, axis=-1, keepdims=True)
            var = jnp.mean((h - mu) ** 2, axis=-1, keepdims=True)
            hn = (h - mu) / jnp.sqrt(var + 1e-5) * g_r[0:1, :] + be_r[0:1, :]
            out_r[...] = jnp.dot(hn, w2_r[...], preferred_element_type=f32) + b2_r[0:1, :]

    return pl.pallas_call(
        body,
        grid=grid,
        in_specs=[
            pl.BlockSpec((BN, H), row),
            pl.BlockSpec((BN, 1), row),
            pl.BlockSpec((BN, 1), row),
            pl.BlockSpec((G, 8), const2),
            pl.BlockSpec((H + 8, 32), const2),
            pl.BlockSpec((8, 32), const2),
            pl.BlockSpec((8, 32), const2),
            pl.BlockSpec((8, 32), const2),
            pl.BlockSpec((32, 1), const2),
            pl.BlockSpec((8, 1), const2),
        ],
        out_specs=pl.BlockSpec((G, 1), const2),
        out_shape=jax.ShapeDtypeStruct((G, 1), f32),
        scratch_shapes=[
            pltpu.VMEM((G, H), f32),
            pltpu.VMEM((G, 1), f32),
        ],
    )(x2, batch2d, ones_col, u, fc1w, fc1b, g3, b3, fc2w, fc2b)


def _b8(b):
    return jnp.broadcast_to(b.reshape(1, -1), (8, b.shape[0]))


def kernel(x, edge_index, edge_attr, u, batch, params):
    p = params
    src = edge_index[0]
    dst = edge_index[1]

    wcat1 = jnp.concatenate([p["conv1a_lin_W"], p["conv1b_lin_W"],
                             p["conv1c_lin_W"]], axis=1)
    bcat1 = jnp.concatenate([p["conv1a_lin_b"], p["conv1b_lin_b"],
                             p["conv1c_lin_b"]], axis=0)
    wc1 = p["emlp1_l2_W"] @ wcat1
    bc1 = p["emlp1_l2_b"] @ wcat1 + bcat1
    wcat2 = jnp.concatenate([p["conv2a_lin_W"], p["conv2b_lin_W"],
                             p["conv2c_lin_W"]], axis=1)
    bcat2 = jnp.concatenate([p["conv2a_lin_b"], p["conv2b_lin_b"],
                             p["conv2c_lin_b"]], axis=0)
    wc2 = p["emlp2_l2_W"] @ wcat2
    bc2 = p["emlp2_l2_b"] @ wcat2 + bcat2

    ea1a, ea1b, ea1c, ea2a, ea2b, ea2c = _edge_transform(
        edge_attr,
        p["emlp1_l1_W"], _b8(p["emlp1_l1_b"]), wc1, _b8(bc1),
        p["emlp2_l1_W"], _b8(p["emlp2_l1_b"]), wc2, _b8(bc2))

    z128 = jnp.zeros((N, F_IN), jnp.float32)
    z64 = jnp.zeros((N, H), jnp.float32)

    pa = _edge_aggregate(x, ea1a, src, dst, z128, F_IN)
    pb = _edge_aggregate(x, ea1b, src, dst, z128, F_IN)
    pc = _edge_aggregate(x, ea1c, src, dst, z128, F_IN)
    x1 = _node_update(
        x, pa, pb, pc,
        [p["conv1a_nn1_W"], p["conv1b_nn1_W"], p["conv1c_nn1_W"]],
        [_b8(p["conv1a_nn1_b"]), _b8(p["conv1b_nn1_b"]), _b8(p["conv1c_nn1_b"])],
        [p["conv1a_nn2_W"], p["conv1b_nn2_W"], p["conv1c_nn2_W"]],
        [_b8(p["conv1a_nn2_b"]), _b8(p["conv1b_nn2_b"]), _b8(p["conv1c_nn2_b"])],
        p["lin1_W"], _b8(p["lin1_b"]), _b8(p["ln1_g"]), _b8(p["ln1_beta"]),
        F_IN)

    qa = _edge_aggregate(x1, ea2a, src, dst, z64, H)
    qb = _edge_aggregate(x1, ea2b, src, dst, z64, H)
    qc = _edge_aggregate(x1, ea2c, src, dst, z64, H)
    x2 = _node_update(
        x1, qa, qb, qc,
        [p["conv2a_nn1_W"], p["conv2b_nn1_W"], p["conv2c_nn1_W"]],
        [_b8(p["conv2a_nn1_b"]), _b8(p["conv2b_nn1_b"]), _b8(p["conv2c_nn1_b"])],
        [p["conv2a_nn2_W"], p["conv2b_nn2_W"], p["conv2c_nn2_W"]],
        [_b8(p["conv2a_nn2_b"]), _b8(p["conv2b_nn2_b"]), _b8(p["conv2c_nn2_b"])],
        p["lin2_W"], _b8(p["lin2_b"]), _b8(p["ln2_g"]), _b8(p["ln2_beta"]),
        H)

    out = _pool_head(
        x2, batch.reshape(N, 1), jnp.ones((N, 1), jnp.float32), u,
        p["fc1_W"], _b8(p["fc1_b"]), _b8(p["ln3_g"]), _b8(p["ln3_beta"]),
        p["fc2_W"], _b8(p["fc2_b"]))
    return out


# speedup vs baseline: 2.4309x; 1.3823x over previous
import functools

import jax
import jax.numpy as jnp
from jax import lax
from jax.experimental import pallas as pl
from jax.experimental.pallas import tpu as pltpu
from jax.experimental.pallas import tpu_sc as plsc

N = 10000
E = 320000
F_IN = 128
H = 64
G = 16

NC = 2
NS = 16
NW = NC * NS
CH = 128
NCHT = E // CH
NCHW = NCHT // NW
NTAIL = NCHT - NCHW * NW

ZROW = 624
ZCNT = 640


def _edge_aggregate(tabs, ea, src, dst, zeros, npack):
    W = H * npack
    mesh = plsc.VectorSubcoreMesh(core_axis_name="c", subcore_axis_name="s")
    utabs, slot2tab = [], []
    for t in tabs:
        if not any(t is ut for ut in utabs):
            utabs.append(t)
        slot2tab.append([i for i, ut in enumerate(utabs) if ut is t][0])
    ngather = len(utabs)

    @functools.partial(
        pl.kernel,
        mesh=mesh,
        compiler_params=pltpu.CompilerParams(use_tc_tiling_on_sc=False),
        out_type=jax.ShapeDtypeStruct((2 * N, W), jnp.float32),
        scratch_types=[
            pltpu.VMEM((CH,), jnp.int32),
            pltpu.VMEM((CH,), jnp.int32),
            [pltpu.VMEM((CH, H), jnp.float32)] * ngather,
            pltpu.VMEM((CH, W), jnp.float32),
            pltpu.VMEM_SHARED((N, W), jnp.float32),
            pltpu.SemaphoreType.DMA,
        ],
    )
    def k(*refs):
        tab_hbms = refs[:ngather]
        ea_hbm, src_hbm, dst_hbm, z_hbm, out_hbm = refs[ngather:ngather + 5]
        src_v, dst_v, xg_vs, m_v, aggr_sh, sem = refs[ngather + 5:]
        cid = lax.axis_index("c")
        sid = lax.axis_index("s")
        wid = cid * NS + sid
        zbase = sid * ZROW

        pltpu.sync_copy(z_hbm.at[pl.ds(zbase, ZCNT)], aggr_sh.at[pl.ds(zbase, ZCNT)])
        plsc.subcore_barrier()

        def process(base):
            pltpu.sync_copy(src_hbm.at[pl.ds(base, CH)], src_v)
            pltpu.sync_copy(dst_hbm.at[pl.ds(base, CH)], dst_v)
            pltpu.sync_copy(ea_hbm.at[pl.ds(base, CH)], m_v)
            for t in range(ngather):
                pltpu.async_copy(tab_hbms[t].at[src_v], xg_vs[t], sem).wait()

            @plsc.parallel_loop(0, CH)
            def _(r):
                for j in range(H // 16):
                    gs = [xg_vs[t][r, pl.ds(j * 16, 16)] for t in range(ngather)]
                    for s in range(npack):
                        sl = pl.ds(s * H + j * 16, 16)
                        m_v[r, sl] = jnp.maximum(m_v[r, sl] + gs[slot2tab[s]], 0.0)

            pltpu.sync_copy(m_v, aggr_sh.at[dst_v], add=True)

        def chunk_body(kk, carry):
            process((wid * NCHW + kk) * CH)
            return carry

        lax.fori_loop(0, NCHW, chunk_body, 0)

        @pl.when(wid < NTAIL)
        def _():
            process((NW * NCHW + wid) * CH)

        plsc.subcore_barrier()
        pltpu.sync_copy(aggr_sh.at[pl.ds(zbase, ZCNT)],
                        out_hbm.at[pl.ds(cid * N + zbase, ZCNT)])

    return k(*utabs, ea, src, dst, zeros)


def _edge_transform(ea, w1a, b1a, wc1, bc1, w1b, b1b, wc2, bc2):
    BE = 3200
    grid = (E // BE,)
    const2 = lambda i: (0, 0)
    row = lambda i: (i, 0)

    def body(ea_ref, w1a_r, b1a_r, wc1_r, bc1_r, w1b_r, b1b_r, wc2_r, bc2_r,
             o_p1, o_p2, o_p3, o_ab, o_c):
        e = ea_ref[...]
        t1 = jnp.maximum(
            jnp.dot(e, w1a_r[...], preferred_element_type=jnp.float32)
            + b1a_r[0:1, :], 0.0)
        z1 = jnp.dot(t1, wc1_r[...], preferred_element_type=jnp.float32) + bc1_r[0:1, :]
        o_p1[...] = jnp.concatenate([z1[:, 0:64], z1[:, 128:192]], axis=1)
        o_p2[...] = jnp.concatenate([z1[:, 64:128], z1[:, 192:256]], axis=1)
        o_p3[...] = z1[:, 256:384]
        t2 = jnp.maximum(
            jnp.dot(e, w1b_r[...], preferred_element_type=jnp.float32)
            + b1b_r[0:1, :], 0.0)
        z2 = jnp.dot(t2, wc2_r[...], preferred_element_type=jnp.float32) + bc2_r[0:1, :]
        o_ab[...] = z2[:, 0:128]
        o_c[...] = z2[:, 128:192]

    f32 = jnp.float32
    return pl.pallas_call(
        body,
        grid=grid,
        in_specs=[
            pl.BlockSpec((BE, 16), row),
            pl.BlockSpec((16, H), const2),
            pl.BlockSpec((8, H), const2),
            pl.BlockSpec((H, 3 * F_IN), const2),
            pl.BlockSpec((8, 3 * F_IN), const2),
            pl.BlockSpec((16, H), const2),
            pl.BlockSpec((8, H), const2),
            pl.BlockSpec((H, 3 * H), const2),
            pl.BlockSpec((8, 3 * H), const2),
        ],
        out_specs=[
            pl.BlockSpec((BE, 2 * H), row),
            pl.BlockSpec((BE, 2 * H), row),
            pl.BlockSpec((BE, 2 * H), row),
            pl.BlockSpec((BE, 2 * H), row),
            pl.BlockSpec((BE, H), row),
        ],
        out_shape=[
            jax.ShapeDtypeStruct((E, 2 * H), f32),
            jax.ShapeDtypeStruct((E, 2 * H), f32),
            jax.ShapeDtypeStruct((E, 2 * H), f32),
            jax.ShapeDtypeStruct((E, 2 * H), f32),
            jax.ShapeDtypeStruct((E, H), f32),
        ],
    )(ea, w1a, b1a, wc1, bc1, w1b, b1b, wc2, bc2)


def _node_update(xin, parts, conv_specs, n1w, n1b, n2w, n2b,
                 l1w, l1b, g, beta, Fin):
    BN = 1000
    grid = (N // BN,)
    row = lambda i: (i, 0)
    shift = lambda i: (i + N // BN, 0)
    const2 = lambda i: (0, 0)
    f32 = jnp.float32
    nparts = len(parts)

    def body(*refs):
        x_r = refs[0]
        prefs = refs[1:1 + 2 * nparts]
        (n1wa, n1wb, n1wc, n1ba, n1bb, n1bc,
         n2wa, n2wb, n2wc, n2ba, n2bb, n2bc,
         l1w_r, l1b_r, g_r, beta_r, out_r) = refs[1 + 2 * nparts:]
        x_b = x_r[...]
        psums = [prefs[2 * ph][...] + prefs[2 * ph + 1][...]
                 for ph in range(nparts)]

        def conv(c, w1, b1, w2, b2):
            segs = [psums[pi][:, off:off + wid] for pi, off, wid in conv_specs[c]]
            h = x_b + (jnp.concatenate(segs, axis=1) if len(segs) > 1 else segs[0])
            t = jnp.maximum(
                jnp.dot(h, w1[...], preferred_element_type=f32) + b1[0:1, :], 0.0)
            o = jnp.dot(t, w2[...], preferred_element_type=f32) + b2[0:1, :]
            return jnp.maximum(o, 0.0)

        cat = jnp.concatenate([
            conv(0, n1wa, n1ba, n2wa, n2ba),
            conv(1, n1wb, n1bb, n2wb, n2bb),
            conv(2, n1wc, n1bc, n2wc, n2bc),
        ], axis=1)
        y = jnp.maximum(
            jnp.dot(cat, l1w_r[...], preferred_element_type=f32) + l1b_r[0:1, :], 0.0)
        m = jnp.mean(y, axis=-1, keepdims=True)
        v = jnp.mean((y - m) ** 2, axis=-1, keepdims=True)
        out_r[...] = (y - m) / jnp.sqrt(v + 1e-5) * g_r[0:1, :] + beta_r[0:1, :]

    in_specs = [pl.BlockSpec((BN, Fin), row)]
    operands = [xin]
    for pt in parts:
        wp = pt.shape[1]
        in_specs += [pl.BlockSpec((BN, wp), row), pl.BlockSpec((BN, wp), shift)]
        operands += [pt, pt]
    in_specs += [pl.BlockSpec((Fin, H), const2)] * 3
    in_specs += [pl.BlockSpec((8, H), const2)] * 3
    in_specs += [pl.BlockSpec((H, H), const2)] * 3
    in_specs += [pl.BlockSpec((8, H), const2)] * 3
    in_specs += [
        pl.BlockSpec((3 * H, H), const2),
        pl.BlockSpec((8, H), const2),
        pl.BlockSpec((8, H), const2),
        pl.BlockSpec((8, H), const2),
    ]
    operands += [n1w[0], n1w[1], n1w[2], n1b[0], n1b[1], n1b[2],
                 n2w[0], n2w[1], n2w[2], n2b[0], n2b[1], n2b[2],
                 l1w, l1b, g, beta]
    return pl.pallas_call(
        body,
        grid=grid,
        in_specs=in_specs,
        out_specs=pl.BlockSpec((BN, H), row),
        out_shape=jax.ShapeDtypeStruct((N, H), f32),
    )(*operands)


def _pool_head(x2, batch2d, ones_col, u, fc1w, fc1b, g3, b3, fc2w, fc2b):
    BN = 1000
    grid = (N // BN,)
    row = lambda i: (i, 0)
    const2 = lambda i: (0, 0)
    f32 = jnp.float32

    def body(x_r, b_r, one_r, u_r, w1_r, b1_r, g_r, be_r, w2_r, b2_r, out_r,
             sums, cnts):
        i = pl.program_id(0)

        @pl.when(i == 0)
        def _():
            sums[...] = jnp.zeros_like(sums)
            cnts[...] = jnp.zeros_like(cnts)

        oh = (b_r[...] == lax.broadcasted_iota(jnp.int32, (1, G), 1)).astype(f32)
        sums[...] += lax.dot_general(oh, x_r[...], (((0,), (0,)), ((), ())),
                                     preferred_element_type=f32)
        cnts[...] += lax.dot_general(oh, one_r[...], (((0,), (0,)), ((), ())),
                                     preferred_element_type=f32)

        @pl.when(i == grid[0] - 1)
        def _():
            mean = sums[...] / jnp.maximum(cnts[...], 1.0)
            xf = jnp.concatenate([mean, u_r[...]], axis=1)
            h = jnp.maximum(
                jnp.dot(xf, w1_r[...], preferred_element_type=f32) + b1_r[0:1, :],
                0.0)
            mu = jnp.mean(h, axis=-1, keepdims=True)
            var = jnp.mean((h - mu) ** 2, axis=-1, keepdims=True)
            hn = (h - mu) / jnp.sqrt(var + 1e-5) * g_r[0:1, :] + be_r[0:1, :]
            out_r[...] = jnp.dot(hn, w2_r[...], preferred_element_type=f32) + b2_r[0:1, :]

    return pl.pallas_call(
        body,
        grid=grid,
        in_specs=[
            pl.BlockSpec((BN, H), row),
            pl.BlockSpec((BN, 1), row),
            pl.BlockSpec((BN, 1), row),
            pl.BlockSpec((G, 8), const2),
            pl.BlockSpec((H + 8, 32), const2),
            pl.BlockSpec((8, 32), const2),
            pl.BlockSpec((8, 32), const2),
            pl.BlockSpec((8, 32), const2),
            pl.BlockSpec((32, 1), const2),
            pl.BlockSpec((8, 1), const2),
        ],
        out_specs=pl.BlockSpec((G, 1), const2),
        out_shape=jax.ShapeDtypeStruct((G, 1), f32),
        scratch_shapes=[
            pltpu.VMEM((G, H), f32),
            pltpu.VMEM((G, 1), f32),
        ],
    )(x2, batch2d, ones_col, u, fc1w, fc1b, g3, b3, fc2w, fc2b)


def _b8(b):
    return jnp.broadcast_to(b.reshape(1, -1), (8, b.shape[0]))


def kernel(x, edge_index, edge_attr, u, batch, params):
    p = params
    src = edge_index[0]
    dst = edge_index[1]

    wcat1 = jnp.concatenate([p["conv1a_lin_W"], p["conv1b_lin_W"],
                             p["conv1c_lin_W"]], axis=1)
    bcat1 = jnp.concatenate([p["conv1a_lin_b"], p["conv1b_lin_b"],
                             p["conv1c_lin_b"]], axis=0)
    wc1 = p["emlp1_l2_W"] @ wcat1
    bc1 = p["emlp1_l2_b"] @ wcat1 + bcat1
    wcat2 = jnp.concatenate([p["conv2a_lin_W"], p["conv2b_lin_W"],
                             p["conv2c_lin_W"]], axis=1)
    bcat2 = jnp.concatenate([p["conv2a_lin_b"], p["conv2b_lin_b"],
                             p["conv2c_lin_b"]], axis=0)
    wc2 = p["emlp2_l2_W"] @ wcat2
    bc2 = p["emlp2_l2_b"] @ wcat2 + bcat2

    ea_p1, ea_p2, ea_p3, ea_ab, ea_c = _edge_transform(
        edge_attr,
        p["emlp1_l1_W"], _b8(p["emlp1_l1_b"]), wc1, _b8(bc1),
        p["emlp2_l1_W"], _b8(p["emlp2_l1_b"]), wc2, _b8(bc2))

    z128 = jnp.zeros((N, 2 * H), jnp.float32)
    z64 = jnp.zeros((N, H), jnp.float32)
    x_lo = lax.slice(x, (0, 0), (N, H))
    x_hi = lax.slice(x, (0, H), (N, F_IN))

    pp1 = _edge_aggregate([x_lo, x_lo], ea_p1, src, dst, z128, 2)
    pp2 = _edge_aggregate([x_hi, x_hi], ea_p2, src, dst, z128, 2)
    pp3 = _edge_aggregate([x_lo, x_hi], ea_p3, src, dst, z128, 2)
    x1 = _node_update(
        x, [pp1, pp2, pp3],
        [[(0, 0, H), (1, 0, H)],
         [(0, H, H), (1, H, H)],
         [(2, 0, 2 * H)]],
        [p["conv1a_nn1_W"], p["conv1b_nn1_W"], p["conv1c_nn1_W"]],
        [_b8(p["conv1a_nn1_b"]), _b8(p["conv1b_nn1_b"]), _b8(p["conv1c_nn1_b"])],
        [p["conv1a_nn2_W"], p["conv1b_nn2_W"], p["conv1c_nn2_W"]],
        [_b8(p["conv1a_nn2_b"]), _b8(p["conv1b_nn2_b"]), _b8(p["conv1c_nn2_b"])],
        p["lin1_W"], _b8(p["lin1_b"]), _b8(p["ln1_g"]), _b8(p["ln1_beta"]),
        F_IN)

    q_ab = _edge_aggregate([x1, x1], ea_ab, src, dst, z128, 2)
    q_c = _edge_aggregate([x1], ea_c, src, dst, z64, 1)
    x2 = _node_update(
        x1, [q_ab, q_c],
        [[(0, 0, H)], [(0, H, H)], [(1, 0, H)]],
        [p["conv2a_nn1_W"], p["conv2b_nn1_W"], p["conv2c_nn1_W"]],
        [_b8(p["conv2a_nn1_b"]), _b8(p["conv2b_nn1_b"]), _b8(p["conv2c_nn1_b"])],
        [p["conv2a_nn2_W"], p["conv2b_nn2_W"], p["conv2c_nn2_W"]],
        [_b8(p["conv2a_nn2_b"]), _b8(p["conv2b_nn2_b"]), _b8(p["conv2c_nn2_b"])],
        p["lin2_W"], _b8(p["lin2_b"]), _b8(p["ln2_g"]), _b8(p["ln2_beta"]),
        H)

    out = _pool_head(
        x2, batch.reshape(N, 1), jnp.ones((N, 1), jnp.float32), u,
        p["fc1_W"], _b8(p["fc1_b"]), _b8(p["ln3_g"]), _b8(p["ln3_beta"]),
        p["fc2_W"], _b8(p["fc2_b"]))
    return out


# speedup vs baseline: 4.0003x; 1.6456x over previous
import functools

import jax
import jax.numpy as jnp
from jax import lax
from jax.experimental import pallas as pl
from jax.experimental.pallas import tpu as pltpu
from jax.experimental.pallas import tpu_sc as plsc

N = 10000
E = 320000
F_IN = 128
H = 64
G = 16

NC = 2
NS = 16
NW = NC * NS
CH = 64
NCHT = E // CH
NCHW = NCHT // NW
NTAIL = NCHT - NCHW * NW

ZROW = 624
ZCNT = 640


def _edge_aggregate(tabs, ea, src, dst, zeros, npack):
    W = H * npack
    mesh = plsc.VectorSubcoreMesh(core_axis_name="c", subcore_axis_name="s")
    utabs, slot2tab = [], []
    for t in tabs:
        if not any(t is ut for ut in utabs):
            utabs.append(t)
        slot2tab.append([i for i, ut in enumerate(utabs) if ut is t][0])
    ngather = len(utabs)
    NB = 3 if ngather == 1 else 2

    @functools.partial(
        pl.kernel,
        mesh=mesh,
        compiler_params=pltpu.CompilerParams(use_tc_tiling_on_sc=False),
        out_type=jax.ShapeDtypeStruct((2 * N, W), jnp.float32),
        scratch_types=[
            [pltpu.VMEM((CH,), jnp.int32)] * NB,
            [pltpu.VMEM((CH,), jnp.int32)] * NB,
            [pltpu.VMEM((CH, H), jnp.float32)] * (NB * ngather),
            [pltpu.VMEM((CH, W), jnp.float32)] * NB,
            pltpu.VMEM_SHARED((N, W), jnp.float32),
            [pltpu.SemaphoreType.DMA] * NB,
            [pltpu.SemaphoreType.DMA] * NB,
            [pltpu.SemaphoreType.DMA] * NB,
            [pltpu.SemaphoreType.DMA] * NB,
        ],
    )
    def k(*refs):
        tab_hbms = refs[:ngather]
        ea_hbm, src_hbm, dst_hbm, z_hbm, out_hbm = refs[ngather:ngather + 5]
        (src_vs, dst_vs, xg_vs, m_vs, aggr_sh,
         sem_meta, sem_ea, sem_g, sem_sc) = refs[ngather + 5:]
        cid = lax.axis_index("c")
        sid = lax.axis_index("s")
        wid = cid * NS + sid
        zbase = sid * ZROW

        pltpu.sync_copy(z_hbm.at[pl.ds(zbase, ZCNT)], aggr_sh.at[pl.ds(zbase, ZCNT)])
        plsc.subcore_barrier()

        def issue_eg(kk, b):
            base = (wid * NCHW + kk) * CH
            c1 = pltpu.async_copy(src_hbm.at[pl.ds(base, CH)], src_vs[b], sem_meta[b])
            c2 = pltpu.async_copy(dst_hbm.at[pl.ds(base, CH)], dst_vs[b], sem_meta[b])
            pltpu.async_copy(ea_hbm.at[pl.ds(base, CH)], m_vs[b], sem_ea[b])
            c1.wait()
            c2.wait()
            for t in range(ngather):
                pltpu.async_copy(tab_hbms[t].at[src_vs[b]], xg_vs[t * NB + b],
                                 sem_g[b])

        def wait_g_ea(b):
            for t in range(ngather):
                pltpu.make_async_copy(tab_hbms[t].at[src_vs[b]],
                                      xg_vs[t * NB + b], sem_g[b]).wait()
            pltpu.make_async_copy(ea_hbm.at[pl.ds(0, CH)], m_vs[b],
                                  sem_ea[b]).wait()

        def compute(b):
            @plsc.parallel_loop(0, CH, unroll=2)
            def _(r):
                for j in range(H // 16):
                    gs = [xg_vs[t * NB + b][r, pl.ds(j * 16, 16)]
                          for t in range(ngather)]
                    for s in range(npack):
                        sl = pl.ds(s * H + j * 16, 16)
                        m_vs[b][r, sl] = jnp.maximum(
                            m_vs[b][r, sl] + gs[slot2tab[s]], 0.0)

        for b in range(NB):
            issue_eg(b, b)

        def pipe_body(m, carry):
            for b in range(NB):
                wait_g_ea(b)
                compute(b)
                pltpu.async_copy(m_vs[b], aggr_sh.at[dst_vs[b]], sem_sc[b],
                                 add=True)
            for b in range(NB):
                pltpu.make_async_copy(m_vs[b], aggr_sh.at[dst_vs[b]],
                                      sem_sc[b]).wait()
                issue_eg(NB * (m + 1) + b, b)
            return carry

        lax.fori_loop(0, NCHW // NB, pipe_body, 0)
        for b in range(NB):
            wait_g_ea(b)

        @pl.when(wid < NTAIL)
        def _():
            base = (NW * NCHW + wid) * CH
            pltpu.sync_copy(src_hbm.at[pl.ds(base, CH)], src_vs[0])
            pltpu.sync_copy(dst_hbm.at[pl.ds(base, CH)], dst_vs[0])
            pltpu.sync_copy(ea_hbm.at[pl.ds(base, CH)], m_vs[0])
            for t in range(ngather):
                pltpu.async_copy(tab_hbms[t].at[src_vs[0]], xg_vs[t * NB],
                                 sem_g[0]).wait()
            compute(0)
            pltpu.sync_copy(m_vs[0], aggr_sh.at[dst_vs[0]], add=True)

        plsc.subcore_barrier()
        pltpu.sync_copy(aggr_sh.at[pl.ds(zbase, ZCNT)],
                        out_hbm.at[pl.ds(cid * N + zbase, ZCNT)])

    return k(*utabs, ea, src, dst, zeros)


def _edge_transform(ea, w1a, b1a, wc1, bc1, w1b, b1b, wc2, bc2):
    BE = 3200
    grid = (E // BE,)
    const2 = lambda i: (0, 0)
    row = lambda i: (i, 0)

    def body(ea_ref, w1a_r, b1a_r, wc1_r, bc1_r, w1b_r, b1b_r, wc2_r, bc2_r,
             o_p1, o_p2, o_p3, o_ab, o_c):
        e = ea_ref[...]
        t1 = jnp.maximum(
            jnp.dot(e, w1a_r[...], preferred_element_type=jnp.float32)
            + b1a_r[0:1, :], 0.0)
        z1 = jnp.dot(t1, wc1_r[...], preferred_element_type=jnp.float32) + bc1_r[0:1, :]
        o_p1[...] = jnp.concatenate([z1[:, 0:64], z1[:, 128:192]], axis=1)
        o_p2[...] = jnp.concatenate([z1[:, 64:128], z1[:, 192:256]], axis=1)
        o_p3[...] = z1[:, 256:384]
        t2 = jnp.maximum(
            jnp.dot(e, w1b_r[...], preferred_element_type=jnp.float32)
            + b1b_r[0:1, :], 0.0)
        z2 = jnp.dot(t2, wc2_r[...], preferred_element_type=jnp.float32) + bc2_r[0:1, :]
        o_ab[...] = z2[:, 0:128]
        o_c[...] = z2[:, 128:192]

    f32 = jnp.float32
    return pl.pallas_call(
        body,
        grid=grid,
        in_specs=[
            pl.BlockSpec((BE, 16), row),
            pl.BlockSpec((16, H), const2),
            pl.BlockSpec((8, H), const2),
            pl.BlockSpec((H, 3 * F_IN), const2),
            pl.BlockSpec((8, 3 * F_IN), const2),
            pl.BlockSpec((16, H), const2),
            pl.BlockSpec((8, H), const2),
            pl.BlockSpec((H, 3 * H), const2),
            pl.BlockSpec((8, 3 * H), const2),
        ],
        out_specs=[
            pl.BlockSpec((BE, 2 * H), row),
            pl.BlockSpec((BE, 2 * H), row),
            pl.BlockSpec((BE, 2 * H), row),
            pl.BlockSpec((BE, 2 * H), row),
            pl.BlockSpec((BE, H), row),
        ],
        out_shape=[
            jax.ShapeDtypeStruct((E, 2 * H), f32),
            jax.ShapeDtypeStruct((E, 2 * H), f32),
            jax.ShapeDtypeStruct((E, 2 * H), f32),
            jax.ShapeDtypeStruct((E, 2 * H), f32),
            jax.ShapeDtypeStruct((E, H), f32),
        ],
    )(ea, w1a, b1a, wc1, bc1, w1b, b1b, wc2, bc2)


def _node_update(xin, parts, conv_specs, n1w, n1b, n2w, n2b,
                 l1w, l1b, g, beta, Fin):
    BN = 1000
    grid = (N // BN,)
    row = lambda i: (i, 0)
    shift = lambda i: (i + N // BN, 0)
    const2 = lambda i: (0, 0)
    f32 = jnp.float32
    nparts = len(parts)

    def body(*refs):
        x_r = refs[0]
        prefs = refs[1:1 + 2 * nparts]
        (n1wa, n1wb, n1wc, n1ba, n1bb, n1bc,
         n2wa, n2wb, n2wc, n2ba, n2bb, n2bc,
         l1w_r, l1b_r, g_r, beta_r, out_r) = refs[1 + 2 * nparts:]
        x_b = x_r[...]
        psums = [prefs[2 * ph][...] + prefs[2 * ph + 1][...]
                 for ph in range(nparts)]

        def conv(c, w1, b1, w2, b2):
            segs = [psums[pi][:, off:off + wid] for pi, off, wid in conv_specs[c]]
            h = x_b + (jnp.concatenate(segs, axis=1) if len(segs) > 1 else segs[0])
            t = jnp.maximum(
                jnp.dot(h, w1[...], preferred_element_type=f32) + b1[0:1, :], 0.0)
            o = jnp.dot(t, w2[...], preferred_element_type=f32) + b2[0:1, :]
            return jnp.maximum(o, 0.0)

        cat = jnp.concatenate([
            conv(0, n1wa, n1ba, n2wa, n2ba),
            conv(1, n1wb, n1bb, n2wb, n2bb),
            conv(2, n1wc, n1bc, n2wc, n2bc),
        ], axis=1)
        y = jnp.maximum(
            jnp.dot(cat, l1w_r[...], preferred_element_type=f32) + l1b_r[0:1, :], 0.0)
        m = jnp.mean(y, axis=-1, keepdims=True)
        v = jnp.mean((y - m) ** 2, axis=-1, keepdims=True)
        out_r[...] = (y - m) / jnp.sqrt(v + 1e-5) * g_r[0:1, :] + beta_r[0:1, :]

    in_specs = [pl.BlockSpec((BN, Fin), row)]
    operands = [xin]
    for pt in parts:
        wp = pt.shape[1]
        in_specs += [pl.BlockSpec((BN, wp), row), pl.BlockSpec((BN, wp), shift)]
        operands += [pt, pt]
    in_specs += [pl.BlockSpec((Fin, H), const2)] * 3
    in_specs += [pl.BlockSpec((8, H), const2)] * 3
    in_specs += [pl.BlockSpec((H, H), const2)] * 3
    in_specs += [pl.BlockSpec((8, H), const2)] * 3
    in_specs += [
        pl.BlockSpec((3 * H, H), const2),
        pl.BlockSpec((8, H), const2),
        pl.BlockSpec((8, H), const2),
        pl.BlockSpec((8, H), const2),
    ]
    operands += [n1w[0], n1w[1], n1w[2], n1b[0], n1b[1], n1b[2],
                 n2w[0], n2w[1], n2w[2], n2b[0], n2b[1], n2b[2],
                 l1w, l1b, g, beta]
    return pl.pallas_call(
        body,
        grid=grid,
        in_specs=in_specs,
        out_specs=pl.BlockSpec((BN, H), row),
        out_shape=jax.ShapeDtypeStruct((N, H), f32),
    )(*operands)


def _pool_head(x2, batch2d, ones_col, u, fc1w, fc1b, g3, b3, fc2w, fc2b):
    BN = 1000
    grid = (N // BN,)
    row = lambda i: (i, 0)
    const2 = lambda i: (0, 0)
    f32 = jnp.float32

    def body(x_r, b_r, one_r, u_r, w1_r, b1_r, g_r, be_r, w2_r, b2_r, out_r,
             sums, cnts):
        i = pl.program_id(0)

        @pl.when(i == 0)
        def _():
            sums[...] = jnp.zeros_like(sums)
            cnts[...] = jnp.zeros_like(cnts)

        oh = (b_r[...] == lax.broadcasted_iota(jnp.int32, (1, G), 1)).astype(f32)
        sums[...] += lax.dot_general(oh, x_r[...], (((0,), (0,)), ((), ())),
                                     preferred_element_type=f32)
        cnts[...] += lax.dot_general(oh, one_r[...], (((0,), (0,)), ((), ())),
                                     preferred_element_type=f32)

        @pl.when(i == grid[0] - 1)
        def _():
            mean = sums[...] / jnp.maximum(cnts[...], 1.0)
            xf = jnp.concatenate([mean, u_r[...]], axis=1)
            h = jnp.maximum(
                jnp.dot(xf, w1_r[...], preferred_element_type=f32) + b1_r[0:1, :],
                0.0)
            mu = jnp.mean(h, axis=-1, keepdims=True)
            var = jnp.mean((h - mu) ** 2, axis=-1, keepdims=True)
            hn = (h - mu) / jnp.sqrt(var + 1e-5) * g_r[0:1, :] + be_r[0:1, :]
            out_r[...] = jnp.dot(hn, w2_r[...], preferred_element_type=f32) + b2_r[0:1, :]

    return pl.pallas_call(
        body,
        grid=grid,
        in_specs=[
            pl.BlockSpec((BN, H), row),
            pl.BlockSpec((BN, 1), row),
            pl.BlockSpec((BN, 1), row),
            pl.BlockSpec((G, 8), const2),
            pl.BlockSpec((H + 8, 32), const2),
            pl.BlockSpec((8, 32), const2),
            pl.BlockSpec((8, 32), const2),
            pl.BlockSpec((8, 32), const2),
            pl.BlockSpec((32, 1), const2),
            pl.BlockSpec((8, 1), const2),
        ],
        out_specs=pl.BlockSpec((G, 1), const2),
        out_shape=jax.ShapeDtypeStruct((G, 1), f32),
        scratch_shapes=[
            pltpu.VMEM((G, H), f32),
            pltpu.VMEM((G, 1), f32),
        ],
    )(x2, batch2d, ones_col, u, fc1w, fc1b, g3, b3, fc2w, fc2b)


def _b8(b):
    return jnp.broadcast_to(b.reshape(1, -1), (8, b.shape[0]))


def kernel(x, edge_index, edge_attr, u, batch, params):
    p = params
    src = edge_index[0]
    dst = edge_index[1]

    wcat1 = jnp.concatenate([p["conv1a_lin_W"], p["conv1b_lin_W"],
                             p["conv1c_lin_W"]], axis=1)
    bcat1 = jnp.concatenate([p["conv1a_lin_b"], p["conv1b_lin_b"],
                             p["conv1c_lin_b"]], axis=0)
    wc1 = p["emlp1_l2_W"] @ wcat1
    bc1 = p["emlp1_l2_b"] @ wcat1 + bcat1
    wcat2 = jnp.concatenate([p["conv2a_lin_W"], p["conv2b_lin_W"],
                             p["conv2c_lin_W"]], axis=1)
    bcat2 = jnp.concatenate([p["conv2a_lin_b"], p["conv2b_lin_b"],
                             p["conv2c_lin_b"]], axis=0)
    wc2 = p["emlp2_l2_W"] @ wcat2
    bc2 = p["emlp2_l2_b"] @ wcat2 + bcat2

    ea_p1, ea_p2, ea_p3, ea_ab, ea_c = _edge_transform(
        edge_attr,
        p["emlp1_l1_W"], _b8(p["emlp1_l1_b"]), wc1, _b8(bc1),
        p["emlp2_l1_W"], _b8(p["emlp2_l1_b"]), wc2, _b8(bc2))

    z128 = jnp.zeros((N, 2 * H), jnp.float32)
    z64 = jnp.zeros((N, H), jnp.float32)
    x_lo = lax.slice(x, (0, 0), (N, H))
    x_hi = lax.slice(x, (0, H), (N, F_IN))

    pp1 = _edge_aggregate([x_lo, x_lo], ea_p1, src, dst, z128, 2)
    pp2 = _edge_aggregate([x_hi, x_hi], ea_p2, src, dst, z128, 2)
    pp3 = _edge_aggregate([x_lo, x_hi], ea_p3, src, dst, z128, 2)
    x1 = _node_update(
        x, [pp1, pp2, pp3],
        [[(0, 0, H), (1, 0, H)],
         [(0, H, H), (1, H, H)],
         [(2, 0, 2 * H)]],
        [p["conv1a_nn1_W"], p["conv1b_nn1_W"], p["conv1c_nn1_W"]],
        [_b8(p["conv1a_nn1_b"]), _b8(p["conv1b_nn1_b"]), _b8(p["conv1c_nn1_b"])],
        [p["conv1a_nn2_W"], p["conv1b_nn2_W"], p["conv1c_nn2_W"]],
        [_b8(p["conv1a_nn2_b"]), _b8(p["conv1b_nn2_b"]), _b8(p["conv1c_nn2_b"])],
        p["lin1_W"], _b8(p["lin1_b"]), _b8(p["ln1_g"]), _b8(p["ln1_beta"]),
        F_IN)

    q_ab = _edge_aggregate([x1, x1], ea_ab, src, dst, z128, 2)
    q_c = _edge_aggregate([x1], ea_c, src, dst, z64, 1)
    x2 = _node_update(
        x1, [q_ab, q_c],
        [[(0, 0, H)], [(0, H, H)], [(1, 0, H)]],
        [p["conv2a_nn1_W"], p["conv2b_nn1_W"], p["conv2c_nn1_W"]],
        [_b8(p["conv2a_nn1_b"]), _b8(p["conv2b_nn1_b"]), _b8(p["conv2c_nn1_b"])],
        [p["conv2a_nn2_W"], p["conv2b_nn2_W"], p["conv2c_nn2_W"]],
        [_b8(p["conv2a_nn2_b"]), _b8(p["conv2b_nn2_b"]), _b8(p["conv2c_nn2_b"])],
        p["lin2_W"], _b8(p["lin2_b"]), _b8(p["ln2_g"]), _b8(p["ln2_beta"]),
        H)

    out = _pool_head(
        x2, batch.reshape(N, 1), jnp.ones((N, 1), jnp.float32), u,
        p["fc1_W"], _b8(p["fc1_b"]), _b8(p["ln3_g"]), _b8(p["ln3_beta"]),
        p["fc2_W"], _b8(p["fc2_b"]))
    return out


# speedup vs baseline: 4.0025x; 1.0006x over previous
import functools

import jax
import jax.numpy as jnp
from jax import lax
from jax.experimental import pallas as pl
from jax.experimental.pallas import tpu as pltpu
from jax.experimental.pallas import tpu_sc as plsc

N = 10000
E = 320000
F_IN = 128
H = 64
G = 16

NC = 2
NS = 16
NW = NC * NS
CH = 64
NCHT = E // CH
NCHW = NCHT // NW
NTAIL = NCHT - NCHW * NW

ZROW = 624
ZCNT = 640


def _edge_aggregate(tabs, ea, src, dst, zeros, npack):
    W = H * npack
    mesh = plsc.VectorSubcoreMesh(core_axis_name="c", subcore_axis_name="s")
    utabs, slot2tab = [], []
    for t in tabs:
        if not any(t is ut for ut in utabs):
            utabs.append(t)
        slot2tab.append([i for i, ut in enumerate(utabs) if ut is t][0])
    ngather = len(utabs)
    NB = 3 if ngather == 1 else 2

    @functools.partial(
        pl.kernel,
        mesh=mesh,
        compiler_params=pltpu.CompilerParams(use_tc_tiling_on_sc=False),
        out_type=jax.ShapeDtypeStruct((2 * N, W), jnp.float32),
        scratch_types=[
            [pltpu.VMEM((CH,), jnp.int32)] * NB,
            [pltpu.VMEM((CH,), jnp.int32)] * NB,
            [pltpu.VMEM((CH, H), jnp.float32)] * (NB * ngather),
            [pltpu.VMEM((CH, W), jnp.float32)] * NB,
            pltpu.VMEM_SHARED((N, W), jnp.float32),
            [pltpu.SemaphoreType.DMA] * NB,
            [pltpu.SemaphoreType.DMA] * NB,
            [pltpu.SemaphoreType.DMA] * NB,
            [pltpu.SemaphoreType.DMA] * NB,
        ],
    )
    def k(*refs):
        tab_hbms = refs[:ngather]
        ea_hbm, src_hbm, dst_hbm, z_hbm, out_hbm = refs[ngather:ngather + 5]
        (src_vs, dst_vs, xg_vs, m_vs, aggr_sh,
         sem_meta, sem_ea, sem_g, sem_sc) = refs[ngather + 5:]
        cid = lax.axis_index("c")
        sid = lax.axis_index("s")
        wid = cid * NS + sid
        zbase = sid * ZROW

        pltpu.sync_copy(z_hbm.at[pl.ds(zbase, ZCNT)], aggr_sh.at[pl.ds(zbase, ZCNT)])
        plsc.subcore_barrier()

        def issue_eg(kk, b):
            base = (wid * NCHW + kk) * CH
            c1 = pltpu.async_copy(src_hbm.at[pl.ds(base, CH)], src_vs[b], sem_meta[b])
            c2 = pltpu.async_copy(dst_hbm.at[pl.ds(base, CH)], dst_vs[b], sem_meta[b])
            pltpu.async_copy(ea_hbm.at[pl.ds(base, CH)], m_vs[b], sem_ea[b])
            c1.wait()
            c2.wait()
            for t in range(ngather):
                pltpu.async_copy(tab_hbms[t].at[src_vs[b]], xg_vs[t * NB + b],
                                 sem_g[b])

        def wait_g_ea(b):
            for t in range(ngather):
                pltpu.make_async_copy(tab_hbms[t].at[src_vs[b]],
                                      xg_vs[t * NB + b], sem_g[b]).wait()
            pltpu.make_async_copy(ea_hbm.at[pl.ds(0, CH)], m_vs[b],
                                  sem_ea[b]).wait()

        def compute(b):
            @plsc.parallel_loop(0, CH, unroll=4)
            def _(r):
                for j in range(H // 16):
                    gs = [xg_vs[t * NB + b][r, pl.ds(j * 16, 16)]
                          for t in range(ngather)]
                    for s in range(npack):
                        sl = pl.ds(s * H + j * 16, 16)
                        m_vs[b][r, sl] = jnp.maximum(
                            m_vs[b][r, sl] + gs[slot2tab[s]], 0.0)

        for b in range(NB):
            issue_eg(b, b)

        def pipe_body(m, carry):
            for b in range(NB):
                wait_g_ea(b)
                compute(b)
                pltpu.async_copy(m_vs[b], aggr_sh.at[dst_vs[b]], sem_sc[b],
                                 add=True)
            for b in range(NB):
                pltpu.make_async_copy(m_vs[b], aggr_sh.at[dst_vs[b]],
                                      sem_sc[b]).wait()
                issue_eg(NB * (m + 1) + b, b)
            return carry

        lax.fori_loop(0, NCHW // NB, pipe_body, 0)
        for b in range(NB):
            wait_g_ea(b)

        @pl.when(wid < NTAIL)
        def _():
            base = (NW * NCHW + wid) * CH
            pltpu.sync_copy(src_hbm.at[pl.ds(base, CH)], src_vs[0])
            pltpu.sync_copy(dst_hbm.at[pl.ds(base, CH)], dst_vs[0])
            pltpu.sync_copy(ea_hbm.at[pl.ds(base, CH)], m_vs[0])
            for t in range(ngather):
                pltpu.async_copy(tab_hbms[t].at[src_vs[0]], xg_vs[t * NB],
                                 sem_g[0]).wait()
            compute(0)
            pltpu.sync_copy(m_vs[0], aggr_sh.at[dst_vs[0]], add=True)

        plsc.subcore_barrier()
        pltpu.sync_copy(aggr_sh.at[pl.ds(zbase, ZCNT)],
                        out_hbm.at[pl.ds(cid * N + zbase, ZCNT)])

    return k(*utabs, ea, src, dst, zeros)


def _edge_transform(ea, w1a, b1a, wc1, bc1, w1b, b1b, wc2, bc2):
    BE = 3200
    grid = (E // BE,)
    const2 = lambda i: (0, 0)
    row = lambda i: (i, 0)

    def body(ea_ref, w1a_r, b1a_r, wc1_r, bc1_r, w1b_r, b1b_r, wc2_r, bc2_r,
             o_p1, o_p2, o_p3, o_ab, o_c):
        e = ea_ref[...]
        t1 = jnp.maximum(
            jnp.dot(e, w1a_r[...], preferred_element_type=jnp.float32)
            + b1a_r[0:1, :], 0.0)
        z1 = jnp.dot(t1, wc1_r[...], preferred_element_type=jnp.float32) + bc1_r[0:1, :]
        o_p1[...] = jnp.concatenate([z1[:, 0:64], z1[:, 128:192]], axis=1)
        o_p2[...] = jnp.concatenate([z1[:, 64:128], z1[:, 192:256]], axis=1)
        o_p3[...] = z1[:, 256:384]
        t2 = jnp.maximum(
            jnp.dot(e, w1b_r[...], preferred_element_type=jnp.float32)
            + b1b_r[0:1, :], 0.0)
        z2 = jnp.dot(t2, wc2_r[...], preferred_element_type=jnp.float32) + bc2_r[0:1, :]
        o_ab[...] = z2[:, 0:128]
        o_c[...] = z2[:, 128:192]

    f32 = jnp.float32
    return pl.pallas_call(
        body,
        grid=grid,
        in_specs=[
            pl.BlockSpec((BE, 16), row),
            pl.BlockSpec((16, H), const2),
            pl.BlockSpec((8, H), const2),
            pl.BlockSpec((H, 3 * F_IN), const2),
            pl.BlockSpec((8, 3 * F_IN), const2),
            pl.BlockSpec((16, H), const2),
            pl.BlockSpec((8, H), const2),
            pl.BlockSpec((H, 3 * H), const2),
            pl.BlockSpec((8, 3 * H), const2),
        ],
        out_specs=[
            pl.BlockSpec((BE, 2 * H), row),
            pl.BlockSpec((BE, 2 * H), row),
            pl.BlockSpec((BE, 2 * H), row),
            pl.BlockSpec((BE, 2 * H), row),
            pl.BlockSpec((BE, H), row),
        ],
        out_shape=[
            jax.ShapeDtypeStruct((E, 2 * H), f32),
            jax.ShapeDtypeStruct((E, 2 * H), f32),
            jax.ShapeDtypeStruct((E, 2 * H), f32),
            jax.ShapeDtypeStruct((E, 2 * H), f32),
            jax.ShapeDtypeStruct((E, H), f32),
        ],
    )(ea, w1a, b1a, wc1, bc1, w1b, b1b, wc2, bc2)


def _node_update(xin, parts, conv_specs, n1w, n1b, n2w, n2b,
                 l1w, l1b, g, beta, Fin):
    BN = 1000
    grid = (N // BN,)
    row = lambda i: (i, 0)
    shift = lambda i: (i + N // BN, 0)
    const2 = lambda i: (0, 0)
    f32 = jnp.float32
    nparts = len(parts)

    def body(*refs):
        x_r = refs[0]
        prefs = refs[1:1 + 2 * nparts]
        (n1wa, n1wb, n1wc, n1ba, n1bb, n1bc,
         n2wa, n2wb, n2wc, n2ba, n2bb, n2bc,
         l1w_r, l1b_r, g_r, beta_r, out_r) = refs[1 + 2 * nparts:]
        x_b = x_r[...]
        psums = [prefs[2 * ph][...] + prefs[2 * ph + 1][...]
                 for ph in range(nparts)]

        def conv(c, w1, b1, w2, b2):
            segs = [psums[pi][:, off:off + wid] for pi, off, wid in conv_specs[c]]
            h = x_b + (jnp.concatenate(segs, axis=1) if len(segs) > 1 else segs[0])
            t = jnp.maximum(
                jnp.dot(h, w1[...], preferred_element_type=f32) + b1[0:1, :], 0.0)
            o = jnp.dot(t, w2[...], preferred_element_type=f32) + b2[0:1, :]
            return jnp.maximum(o, 0.0)

        cat = jnp.concatenate([
            conv(0, n1wa, n1ba, n2wa, n2ba),
            conv(1, n1wb, n1bb, n2wb, n2bb),
            conv(2, n1wc, n1bc, n2wc, n2bc),
        ], axis=1)
        y = jnp.maximum(
            jnp.dot(cat, l1w_r[...], preferred_element_type=f32) + l1b_r[0:1, :], 0.0)
        m = jnp.mean(y, axis=-1, keepdims=True)
        v = jnp.mean((y - m) ** 2, axis=-1, keepdims=True)
        out_r[...] = (y - m) / jnp.sqrt(v + 1e-5) * g_r[0:1, :] + beta_r[0:1, :]

    in_specs = [pl.BlockSpec((BN, Fin), row)]
    operands = [xin]
    for pt in parts:
        wp = pt.shape[1]
        in_specs += [pl.BlockSpec((BN, wp), row), pl.BlockSpec((BN, wp), shift)]
        operands += [pt, pt]
    in_specs += [pl.BlockSpec((Fin, H), const2)] * 3
    in_specs += [pl.BlockSpec((8, H), const2)] * 3
    in_specs += [pl.BlockSpec((H, H), const2)] * 3
    in_specs += [pl.BlockSpec((8, H), const2)] * 3
    in_specs += [
        pl.BlockSpec((3 * H, H), const2),
        pl.BlockSpec((8, H), const2),
        pl.BlockSpec((8, H), const2),
        pl.BlockSpec((8, H), const2),
    ]
    operands += [n1w[0], n1w[1], n1w[2], n1b[0], n1b[1], n1b[2],
                 n2w[0], n2w[1], n2w[2], n2b[0], n2b[1], n2b[2],
                 l1w, l1b, g, beta]
    return pl.pallas_call(
        body,
        grid=grid,
        in_specs=in_specs,
        out_specs=pl.BlockSpec((BN, H), row),
        out_shape=jax.ShapeDtypeStruct((N, H), f32),
    )(*operands)


def _pool_head(x2, batch2d, ones_col, u, fc1w, fc1b, g3, b3, fc2w, fc2b):
    BN = 1000
    grid = (N // BN,)
    row = lambda i: (i, 0)
    const2 = lambda i: (0, 0)
    f32 = jnp.float32

    def body(x_r, b_r, one_r, u_r, w1_r, b1_r, g_r, be_r, w2_r, b2_r, out_r,
             sums, cnts):
        i = pl.program_id(0)

        @pl.when(i == 0)
        def _():
            sums[...] = jnp.zeros_like(sums)
            cnts[...] = jnp.zeros_like(cnts)

        oh = (b_r[...] == lax.broadcasted_iota(jnp.int32, (1, G), 1)).astype(f32)
        sums[...] += lax.dot_general(oh, x_r[...], (((0,), (0,)), ((), ())),
                                     preferred_element_type=f32)
        cnts[...] += lax.dot_general(oh, one_r[...], (((0,), (0,)), ((), ())),
                                     preferred_element_type=f32)

        @pl.when(i == grid[0] - 1)
        def _():
            mean = sums[...] / jnp.maximum(cnts[...], 1.0)
            xf = jnp.concatenate([mean, u_r[...]], axis=1)
            h = jnp.maximum(
                jnp.dot(xf, w1_r[...], preferred_element_type=f32) + b1_r[0:1, :],
                0.0)
            mu = jnp.mean(h, axis=-1, keepdims=True)
            var = jnp.mean((h - mu) ** 2, axis=-1, keepdims=True)
            hn = (h - mu) / jnp.sqrt(var + 1e-5) * g_r[0:1, :] + be_r[0:1, :]
            out_r[...] = jnp.dot(hn, w2_r[...], preferred_element_type=f32) + b2_r[0:1, :]

    return pl.pallas_call(
        body,
        grid=grid,
        in_specs=[
            pl.BlockSpec((BN, H), row),
            pl.BlockSpec((BN, 1), row),
            pl.BlockSpec((BN, 1), row),
            pl.BlockSpec((G, 8), const2),
            pl.BlockSpec((H + 8, 32), const2),
            pl.BlockSpec((8, 32), const2),
            pl.BlockSpec((8, 32), const2),
            pl.BlockSpec((8, 32), const2),
            pl.BlockSpec((32, 1), const2),
            pl.BlockSpec((8, 1), const2),
        ],
        out_specs=pl.BlockSpec((G, 1), const2),
        out_shape=jax.ShapeDtypeStruct((G, 1), f32),
        scratch_shapes=[
            pltpu.VMEM((G, H), f32),
            pltpu.VMEM((G, 1), f32),
        ],
    )(x2, batch2d, ones_col, u, fc1w, fc1b, g3, b3, fc2w, fc2b)


def _b8(b):
    return jnp.broadcast_to(b.reshape(1, -1), (8, b.shape[0]))


def kernel(x, edge_index, edge_attr, u, batch, params):
    p = params
    src = edge_index[0]
    dst = edge_index[1]

    wcat1 = jnp.concatenate([p["conv1a_lin_W"], p["conv1b_lin_W"],
                             p["conv1c_lin_W"]], axis=1)
    bcat1 = jnp.concatenate([p["conv1a_lin_b"], p["conv1b_lin_b"],
                             p["conv1c_lin_b"]], axis=0)
    wc1 = p["emlp1_l2_W"] @ wcat1
    bc1 = p["emlp1_l2_b"] @ wcat1 + bcat1
    wcat2 = jnp.concatenate([p["conv2a_lin_W"], p["conv2b_lin_W"],
                             p["conv2c_lin_W"]], axis=1)
    bcat2 = jnp.concatenate([p["conv2a_lin_b"], p["conv2b_lin_b"],
                             p["conv2c_lin_b"]], axis=0)
    wc2 = p["emlp2_l2_W"] @ wcat2
    bc2 = p["emlp2_l2_b"] @ wcat2 + bcat2

    ea_p1, ea_p2, ea_p3, ea_ab, ea_c = _edge_transform(
        edge_attr,
        p["emlp1_l1_W"], _b8(p["emlp1_l1_b"]), wc1, _b8(bc1),
        p["emlp2_l1_W"], _b8(p["emlp2_l1_b"]), wc2, _b8(bc2))

    z128 = jnp.zeros((N, 2 * H), jnp.float32)
    z64 = jnp.zeros((N, H), jnp.float32)
    x_lo = lax.slice(x, (0, 0), (N, H))
    x_hi = lax.slice(x, (0, H), (N, F_IN))

    pp1 = _edge_aggregate([x_lo, x_lo], ea_p1, src, dst, z128, 2)
    pp2 = _edge_aggregate([x_hi, x_hi], ea_p2, src, dst, z128, 2)
    pp3 = _edge_aggregate([x_lo, x_hi], ea_p3, src, dst, z128, 2)
    x1 = _node_update(
        x, [pp1, pp2, pp3],
        [[(0, 0, H), (1, 0, H)],
         [(0, H, H), (1, H, H)],
         [(2, 0, 2 * H)]],
        [p["conv1a_nn1_W"], p["conv1b_nn1_W"], p["conv1c_nn1_W"]],
        [_b8(p["conv1a_nn1_b"]), _b8(p["conv1b_nn1_b"]), _b8(p["conv1c_nn1_b"])],
        [p["conv1a_nn2_W"], p["conv1b_nn2_W"], p["conv1c_nn2_W"]],
        [_b8(p["conv1a_nn2_b"]), _b8(p["conv1b_nn2_b"]), _b8(p["conv1c_nn2_b"])],
        p["lin1_W"], _b8(p["lin1_b"]), _b8(p["ln1_g"]), _b8(p["ln1_beta"]),
        F_IN)

    q_ab = _edge_aggregate([x1, x1], ea_ab, src, dst, z128, 2)
    q_c = _edge_aggregate([x1], ea_c, src, dst, z64, 1)
    x2 = _node_update(
        x1, [q_ab, q_c],
        [[(0, 0, H)], [(0, H, H)], [(1, 0, H)]],
        [p["conv2a_nn1_W"], p["conv2b_nn1_W"], p["conv2c_nn1_W"]],
        [_b8(p["conv2a_nn1_b"]), _b8(p["conv2b_nn1_b"]), _b8(p["conv2c_nn1_b"])],
        [p["conv2a_nn2_W"], p["conv2b_nn2_W"], p["conv2c_nn2_W"]],
        [_b8(p["conv2a_nn2_b"]), _b8(p["conv2b_nn2_b"]), _b8(p["conv2c_nn2_b"])],
        p["lin2_W"], _b8(p["lin2_b"]), _b8(p["ln2_g"]), _b8(p["ln2_beta"]),
        H)

    out = _pool_head(
        x2, batch.reshape(N, 1), jnp.ones((N, 1), jnp.float32), u,
        p["fc1_W"], _b8(p["fc1_b"]), _b8(p["ln3_g"]), _b8(p["ln3_beta"]),
        p["fc2_W"], _b8(p["fc2_b"]))
    return out
